# Initial kernel scaffold; baseline (speedup 1.0000x reference)
import functools

import jax
import jax.numpy as jnp
from jax import lax
from jax.experimental import pallas as pl
from jax.experimental.pallas import tpu as pltpu
from jax.experimental.pallas import tpu_sc as plsc

NUM_EMB = 1000000
DIM = 32
BATCH = 16384
HIST = 20
TOTAL = BATCH * HIST

_info = plsc.get_sparse_core_info()
_NC = _info.num_cores
_NS = _info.num_subcores
_NW = _NC * _NS

_B_PER_W = TOTAL // _NW
_CHUNK = 1024
_NCHUNKS = _B_PER_W // _CHUNK


def _body(idx_hbm, table_hbm, out_hbm, idx_v, rows_v, sem):
    wid = lax.axis_index("s") * _NC + lax.axis_index("c")
    base = wid * _B_PER_W
    pltpu.sync_copy(idx_hbm.at[pl.ds(base, _B_PER_W)], idx_v)
    for c in range(_NCHUNKS):
        idx_sl = idx_v.at[pl.ds(c * _CHUNK, _CHUNK)]
        pltpu.async_copy(table_hbm.at[idx_sl], rows_v, sem).wait()
        pltpu.sync_copy(rows_v, out_hbm.at[pl.ds(base + c * _CHUNK, _CHUNK)])


_gather = pl.kernel(
    _body,
    out_type=jax.ShapeDtypeStruct((TOTAL, DIM), jnp.float32),
    mesh=plsc.VectorSubcoreMesh(core_axis_name="c", subcore_axis_name="s"),
    scratch_types=[
        pltpu.VMEM((_B_PER_W,), jnp.int32),
        pltpu.VMEM((_CHUNK, DIM), jnp.float32),
        pltpu.SemaphoreType.DMA,
    ],
)


@jax.jit
def kernel(token_ids, weights):
    idx = jnp.reshape(token_ids, (TOTAL,)).astype(jnp.int32)
    out = _gather(idx, weights)
    return jnp.reshape(out, (BATCH, HIST, DIM))


# speedup vs baseline: 1.5005x; 1.5005x over previous
import functools

import jax
import jax.numpy as jnp
from jax import lax
from jax.experimental import pallas as pl
from jax.experimental.pallas import tpu as pltpu
from jax.experimental.pallas import tpu_sc as plsc

NUM_EMB = 1000000
DIM = 32
BATCH = 16384
HIST = 20
TOTAL = BATCH * HIST

_info = plsc.get_sparse_core_info()
_NC = _info.num_cores
_NS = _info.num_subcores
_NW = _NC * _NS

_B_PER_W = TOTAL // _NW
_CHUNK = 1024
_NCHUNKS = _B_PER_W // _CHUNK


def _body(idx_hbm, table_hbm, out_hbm, idx_v, rows_v, sem):
    wid = lax.axis_index("s") * _NC + lax.axis_index("c")
    base = wid * _B_PER_W
    pltpu.sync_copy(idx_hbm.at[pl.ds(base, _B_PER_W)], idx_v)
    for c in range(_NCHUNKS):
        idx_sl = idx_v.at[pl.ds(c * _CHUNK, _CHUNK)]
        pltpu.async_copy(table_hbm.at[idx_sl], rows_v, sem).wait()
        pltpu.sync_copy(rows_v, out_hbm.at[pl.ds(base + c * _CHUNK, _CHUNK)])


_gather = pl.kernel(
    _body,
    out_type=jax.ShapeDtypeStruct((TOTAL, DIM), jnp.float32),
    mesh=plsc.VectorSubcoreMesh(core_axis_name="c", subcore_axis_name="s"),
    scratch_types=[
        pltpu.VMEM((_B_PER_W,), jnp.int32),
        pltpu.VMEM((_CHUNK, DIM), jnp.float32),
        pltpu.SemaphoreType.DMA,
    ],
    compiler_params=pltpu.CompilerParams(use_tc_tiling_on_sc=False),
)


@jax.jit
def kernel(token_ids, weights):
    idx = jnp.reshape(token_ids, (TOTAL,)).astype(jnp.int32)
    out = _gather(idx, weights)
    return jnp.reshape(out, (BATCH, HIST, DIM))


# speedup vs baseline: 1.5115x; 1.0073x over previous
import functools

import jax
import jax.numpy as jnp
from jax import lax
from jax.experimental import pallas as pl
from jax.experimental.pallas import tpu as pltpu
from jax.experimental.pallas import tpu_sc as plsc

NUM_EMB = 1000000
DIM = 32
BATCH = 16384
HIST = 20
TOTAL = BATCH * HIST

_info = plsc.get_sparse_core_info()
_NC = _info.num_cores
_NS = _info.num_subcores
_NW = _NC * _NS

_B_PER_W = TOTAL // _NW
_CHUNK = 1024
_NCHUNKS = _B_PER_W // _CHUNK
_NBUF = 3


def _body(idx_hbm, table_hbm, out_hbm, idx_v, rows_v, *sems):
    g_sems = sems[:_NBUF]
    s_sems = sems[_NBUF:]
    wid = lax.axis_index("s") * _NC + lax.axis_index("c")
    base = wid * _B_PER_W
    pltpu.sync_copy(idx_hbm.at[pl.ds(base, _B_PER_W)], idx_v)

    def start_gather(c):
        b = c % _NBUF
        idx_sl = idx_v.at[pl.ds(c * _CHUNK, _CHUNK)]
        return pltpu.async_copy(table_hbm.at[idx_sl], rows_v.at[b], g_sems[b])

    gathers = [start_gather(c) for c in range(min(_NBUF, _NCHUNKS))]
    gathers += [None] * (_NCHUNKS - len(gathers))
    scatters = [None] * _NCHUNKS
    for c in range(_NCHUNKS):
        b = c % _NBUF
        gathers[c].wait()
        scatters[c] = pltpu.async_copy(
            rows_v.at[b], out_hbm.at[pl.ds(base + c * _CHUNK, _CHUNK)], s_sems[b]
        )
        nc = c + _NBUF
        if nc < _NCHUNKS:
            scatters[c].wait()
            gathers[nc] = start_gather(nc)
    for c in range(max(0, _NCHUNKS - _NBUF), _NCHUNKS):
        scatters[c].wait()


_gather = pl.kernel(
    _body,
    out_type=jax.ShapeDtypeStruct((TOTAL, DIM), jnp.float32),
    mesh=plsc.VectorSubcoreMesh(core_axis_name="c", subcore_axis_name="s"),
    scratch_types=[
        pltpu.VMEM((_B_PER_W,), jnp.int32),
        pltpu.VMEM((_NBUF, _CHUNK, DIM), jnp.float32),
    ]
    + [pltpu.SemaphoreType.DMA] * (2 * _NBUF),
    compiler_params=pltpu.CompilerParams(use_tc_tiling_on_sc=False),
)


@jax.jit
def kernel(token_ids, weights):
    idx = jnp.reshape(token_ids, (TOTAL,)).astype(jnp.int32)
    out = _gather(idx, weights)
    return jnp.reshape(out, (BATCH, HIST, DIM))
